# Initial kernel scaffold; baseline (speedup 1.0000x reference)
import jax
import jax.numpy as jnp
from jax.experimental import pallas as pl


def kernel(query_entities, query_relations, obj_entities, ent_table, rel_table):
    raise NotImplementedError("write your pallas kernel here")


# speedup vs baseline: 2.4578x; 2.4578x over previous
import functools

import jax
import jax.numpy as jnp
from jax.experimental import pallas as pl
from jax.experimental.pallas import tpu as pltpu
from jax.experimental.pallas import tpu_sc as plsc

N_CORES = 2
N_SUBCORES = 16
NW = N_CORES * N_SUBCORES
BATCH = 16384
D_MODEL = 128
CHUNK = 128
B_PER_W = BATCH // NW
N_CHUNKS = B_PER_W // CHUNK


def _make_kernel():
  mesh = plsc.VectorSubcoreMesh(
      core_axis_name="c", subcore_axis_name="s",
      num_cores=N_CORES, num_subcores=N_SUBCORES)
  out_type = (jax.ShapeDtypeStruct((BATCH, D_MODEL), jnp.float32),) * 3

  @functools.partial(
      pl.kernel,
      out_type=out_type,
      mesh=mesh,
      scratch_types=[
          pltpu.VMEM((N_CHUNKS, CHUNK), jnp.int32),
          pltpu.VMEM((N_CHUNKS, CHUNK), jnp.int32),
          pltpu.VMEM((N_CHUNKS, CHUNK), jnp.int32),
          pltpu.VMEM((CHUNK, D_MODEL), jnp.float32),
          pltpu.VMEM((CHUNK, D_MODEL), jnp.float32),
          pltpu.SemaphoreType.DMA,
          pltpu.SemaphoreType.DMA,
      ],
  )
  def trans_e_gather(qe_h, qr_h, oe_h, ent_h, rel_h,
                     out_qe, out_qr, out_oe,
                     idx_q, idx_r, idx_o, buf0, buf1, sem0, sem1):
    wid = jax.lax.axis_index("s") * N_CORES + jax.lax.axis_index("c")
    idx_base = wid * N_CHUNKS
    row_base = wid * B_PER_W

    pltpu.sync_copy(qe_h.at[pl.ds(idx_base, N_CHUNKS)], idx_q)
    pltpu.sync_copy(qr_h.at[pl.ds(idx_base, N_CHUNKS)], idx_r)
    pltpu.sync_copy(oe_h.at[pl.ds(idx_base, N_CHUNKS)], idx_o)

    tasks = []
    for idx_ref, tab, out in ((idx_q, ent_h, out_qe),
                              (idx_r, rel_h, out_qr),
                              (idx_o, ent_h, out_oe)):
      for c in range(N_CHUNKS):
        tasks.append((idx_ref.at[c], tab, out.at[pl.ds(row_base + c * CHUNK, CHUNK)]))

    bufs = (buf0, buf1)
    sems = (sem0, sem1)
    pending = pltpu.async_copy(tasks[0][1].at[tasks[0][0]], bufs[0], sems[0])
    for t in range(len(tasks)):
      b = t % 2
      nxt = None
      if t + 1 < len(tasks):
        idx_s, tab, _ = tasks[t + 1]
        nxt = pltpu.async_copy(tab.at[idx_s], bufs[1 - b], sems[1 - b])
      pending.wait()
      pltpu.sync_copy(bufs[b], tasks[t][2])
      pending = nxt

  return trans_e_gather


_KERNEL = _make_kernel()


def kernel(query_entities, query_relations, obj_entities, ent_table, rel_table):
  qe = query_entities.reshape(NW * N_CHUNKS, CHUNK)
  qr = query_relations.reshape(NW * N_CHUNKS, CHUNK)
  oe = obj_entities.reshape(NW * N_CHUNKS, CHUNK)
  return _KERNEL(qe, qr, oe, ent_table, rel_table)
